# Initial kernel scaffold; baseline (speedup 1.0000x reference)
import jax
import jax.numpy as jnp
from jax.experimental import pallas as pl


def kernel(mcc_code, tr_type, amount, seq_lens, W_mcc, W_tr):
    raise NotImplementedError("write your pallas kernel here")


# speedup vs baseline: 2.0207x; 2.0207x over previous
import functools

import jax
import jax.numpy as jnp
from jax import lax
from jax.experimental import pallas as pl
from jax.experimental.pallas import tpu as pltpu
from jax.experimental.pallas import tpu_sc as plsc

B, L = 4096, 200
N = B * L
D = 16
OUTD = 2 * D + 1
NC, NS = 2, 16
NW = NC * NS
PER_W = N // NW
CHUNK = 512
NCHUNK = PER_W // CHUNK
LN2 = 0.6931471805599453


def _log1p_abs_signed(a):
    x = 1.0 + jnp.abs(a)
    u = lax.bitcast_convert_type(x, jnp.int32)
    un = u + (0x3F800000 - 0x3F3504F3)
    k = (un >> 23) - 127
    um = (un & 0x007FFFFF) + 0x3F3504F3
    m = lax.bitcast_convert_type(um, jnp.float32)
    t = (m - 1.0) / (m + 1.0)
    t2 = t * t
    p = 2.0 * t * (1.0 + t2 * (1.0 / 3.0 + t2 * (0.2 + t2 * (1.0 / 7.0 + t2 * (1.0 / 9.0)))))
    logx = k.astype(jnp.float32) * LN2 + p
    return jnp.sign(a) * logx


def _body(mcc_hbm, tr_hbm, amt_hbm, wm_hbm, wt_hbm, out_hbm,
          idx1_v, idx2_v, amt_v, rows1_v, rows2_v, obuf_v, sem):
    wid = lax.axis_index("s") * NC + lax.axis_index("c")
    base = wid * PER_W

    def chunk_body(c, carry):
        off = base + c * CHUNK
        pltpu.sync_copy(mcc_hbm.at[pl.ds(off, CHUNK)], idx1_v)
        pltpu.sync_copy(tr_hbm.at[pl.ds(off, CHUNK)], idx2_v)
        pltpu.sync_copy(amt_hbm.at[pl.ds(off, CHUNK)], amt_v)
        d1 = pltpu.async_copy(wm_hbm.at[idx1_v], rows1_v, sem)
        d2 = pltpu.async_copy(wt_hbm.at[idx2_v], rows2_v, sem)
        d1.wait()
        d2.wait()

        def row16(i, carry2):
            r0 = i * D
            for j in range(D):
                r = r0 + j
                obuf_v[pl.ds(r * OUTD, D)] = rows1_v[r, :]
                obuf_v[pl.ds(r * OUTD + D, D)] = rows2_v[r, :]
            a = amt_v[pl.ds(r0, D)]
            num = _log1p_abs_signed(a)
            idxs = (r0 + lax.iota(jnp.int32, D)) * OUTD + 2 * D
            plsc.store_scatter(obuf_v, [idxs], num)
            return carry2

        lax.fori_loop(0, CHUNK // D, row16, 0, unroll=False)
        pltpu.sync_copy(obuf_v, out_hbm.at[pl.ds(off * OUTD, CHUNK * OUTD)])
        return carry

    lax.fori_loop(0, NCHUNK, chunk_body, 0, unroll=False)


@jax.jit
def _sc_encode(mcc, tr, amt, wm, wt):
    mesh = plsc.VectorSubcoreMesh(core_axis_name="c", subcore_axis_name="s")
    f = pl.kernel(
        _body,
        out_type=jax.ShapeDtypeStruct((N * OUTD,), jnp.float32),
        mesh=mesh,
        compiler_params=pltpu.CompilerParams(
            needs_layout_passes=False, use_tc_tiling_on_sc=False),
        scratch_types=[
            pltpu.VMEM((CHUNK,), jnp.int32),
            pltpu.VMEM((CHUNK,), jnp.int32),
            pltpu.VMEM((CHUNK,), jnp.float32),
            pltpu.VMEM((CHUNK, D), jnp.float32),
            pltpu.VMEM((CHUNK, D), jnp.float32),
            pltpu.VMEM((CHUNK * OUTD,), jnp.float32),
            pltpu.SemaphoreType.DMA,
        ],
    )
    return f(mcc, tr, amt, wm, wt)


def kernel(mcc_code, tr_type, amount, seq_lens, W_mcc, W_tr):
    del seq_lens
    mcc = mcc_code.reshape(-1).astype(jnp.int32)
    tr = tr_type.reshape(-1).astype(jnp.int32)
    amt = amount.reshape(-1)
    out = _sc_encode(mcc, tr, amt, W_mcc, W_tr)
    return out.reshape(B, L, OUTD)


# speedup vs baseline: 2.2498x; 1.1134x over previous
import functools

import jax
import jax.numpy as jnp
from jax import lax
from jax.experimental import pallas as pl
from jax.experimental.pallas import tpu as pltpu
from jax.experimental.pallas import tpu_sc as plsc

B, L = 4096, 200
N = B * L
D = 16
OUTD = 2 * D + 1
NC, NS = 2, 16
NW = NC * NS
PER_W = N // NW
CHUNK = 512
NCHUNK = PER_W // CHUNK
LN2 = 0.6931471805599453


def _log1p_abs_signed(a):
    x = 1.0 + jnp.abs(a)
    u = lax.bitcast_convert_type(x, jnp.int32)
    un = u + (0x3F800000 - 0x3F3504F3)
    k = (un >> 23) - 127
    um = (un & 0x007FFFFF) + 0x3F3504F3
    m = lax.bitcast_convert_type(um, jnp.float32)
    t = (m - 1.0) / (m + 1.0)
    t2 = t * t
    p = 2.0 * t * (1.0 + t2 * (1.0 / 3.0 + t2 * (0.2 + t2 * (1.0 / 7.0 + t2 * (1.0 / 9.0)))))
    logx = k.astype(jnp.float32) * LN2 + p
    return jnp.sign(a) * logx


def _body(mcc_hbm, tr_hbm, amt_hbm, wm_hbm, wt_hbm, out_hbm,
          idxm0, idxm1, idxt0, idxt1, amt0, amt1,
          rows1_0, rows1_1, rows2_0, rows2_1, obuf0, obuf1,
          si0, si1, sg0, sg1, so0, so1):
    wid = lax.axis_index("s") * NC + lax.axis_index("c")
    base = wid * PER_W
    idxm = (idxm0, idxm1)
    idxt = (idxt0, idxt1)
    amtb = (amt0, amt1)
    rows1 = (rows1_0, rows1_1)
    rows2 = (rows2_0, rows2_1)
    obuf = (obuf0, obuf1)
    sem_i = (si0, si1)
    sem_g = (sg0, sg1)
    sem_o = (so0, so1)

    def start_idx(c, b):
        off = base + c * CHUNK
        pltpu.async_copy(mcc_hbm.at[pl.ds(off, CHUNK)], idxm[b], sem_i[b])
        pltpu.async_copy(tr_hbm.at[pl.ds(off, CHUNK)], idxt[b], sem_i[b])
        pltpu.async_copy(amt_hbm.at[pl.ds(off, CHUNK)], amtb[b], sem_i[b])

    def wait_idx(b):
        pltpu.make_async_copy(mcc_hbm.at[pl.ds(0, CHUNK)], idxm[b], sem_i[b]).wait()
        pltpu.make_async_copy(tr_hbm.at[pl.ds(0, CHUNK)], idxt[b], sem_i[b]).wait()
        pltpu.make_async_copy(amt_hbm.at[pl.ds(0, CHUNK)], amtb[b], sem_i[b]).wait()

    def start_gather(b):
        pltpu.async_copy(wm_hbm.at[idxm[b]], rows1[b], sem_g[b])
        pltpu.async_copy(wt_hbm.at[idxt[b]], rows2[b], sem_g[b])

    def wait_gather(b):
        pltpu.make_async_copy(wm_hbm.at[idxm[b]], rows1[b], sem_g[b]).wait()
        pltpu.make_async_copy(wt_hbm.at[idxt[b]], rows2[b], sem_g[b]).wait()

    def start_out(c, b):
        off = base + c * CHUNK
        pltpu.async_copy(obuf[b], out_hbm.at[pl.ds(off * OUTD, CHUNK * OUTD)], sem_o[b])

    def wait_out(b):
        pltpu.make_async_copy(obuf[b], out_hbm.at[pl.ds(0, CHUNK * OUTD)], sem_o[b]).wait()

    def interleave(b):
        r1, r2, ob, am = rows1[b], rows2[b], obuf[b], amtb[b]

        def row16(i, carry2):
            r0 = i * D
            for j in range(D):
                r = r0 + j
                ob[pl.ds(r * OUTD, D)] = r1[r, :]
                ob[pl.ds(r * OUTD + D, D)] = r2[r, :]
            a = am[pl.ds(r0, D)]
            num = _log1p_abs_signed(a)
            idxs = (r0 + lax.iota(jnp.int32, D)) * OUTD + 2 * D
            plsc.store_scatter(ob, [idxs], num)
            return carry2

        lax.fori_loop(0, CHUNK // D, row16, 0, unroll=False)

    def pipe_step(c, b, next_gather, idx_ahead, out_wait):
        wait_gather(b)
        nb = 1 - b
        if next_gather:
            wait_idx(nb)
            start_gather(nb)
        if out_wait:
            wait_out(b)
        interleave(b)
        if idx_ahead:
            start_idx(c + 2, b)
        start_out(c, b)

    start_idx(0, 0)
    start_idx(1, 1)
    wait_idx(0)
    start_gather(0)
    pipe_step(0, 0, True, True, False)
    pipe_step(1, 1, True, True, False)

    def pipe_k(k, carry):
        c = 2 * k
        pipe_step(c, 0, True, True, True)
        pipe_step(c + 1, 1, True, True, True)
        return carry

    lax.fori_loop(1, NCHUNK // 2 - 1, pipe_k, 0, unroll=False)
    pipe_step(NCHUNK - 2, 0, True, False, True)
    pipe_step(NCHUNK - 1, 1, False, False, True)
    wait_out(0)
    wait_out(1)


@jax.jit
def _sc_encode(mcc, tr, amt, wm, wt):
    mesh = plsc.VectorSubcoreMesh(core_axis_name="c", subcore_axis_name="s")
    f = pl.kernel(
        _body,
        out_type=jax.ShapeDtypeStruct((N * OUTD,), jnp.float32),
        mesh=mesh,
        compiler_params=pltpu.CompilerParams(
            needs_layout_passes=False, use_tc_tiling_on_sc=False),
        scratch_types=[
            pltpu.VMEM((CHUNK,), jnp.int32),
            pltpu.VMEM((CHUNK,), jnp.int32),
            pltpu.VMEM((CHUNK,), jnp.int32),
            pltpu.VMEM((CHUNK,), jnp.int32),
            pltpu.VMEM((CHUNK,), jnp.float32),
            pltpu.VMEM((CHUNK,), jnp.float32),
            pltpu.VMEM((CHUNK, D), jnp.float32),
            pltpu.VMEM((CHUNK, D), jnp.float32),
            pltpu.VMEM((CHUNK, D), jnp.float32),
            pltpu.VMEM((CHUNK, D), jnp.float32),
            pltpu.VMEM((CHUNK * OUTD,), jnp.float32),
            pltpu.VMEM((CHUNK * OUTD,), jnp.float32),
            pltpu.SemaphoreType.DMA,
            pltpu.SemaphoreType.DMA,
            pltpu.SemaphoreType.DMA,
            pltpu.SemaphoreType.DMA,
            pltpu.SemaphoreType.DMA,
            pltpu.SemaphoreType.DMA,
        ],
    )
    return f(mcc, tr, amt, wm, wt)


def kernel(mcc_code, tr_type, amount, seq_lens, W_mcc, W_tr):
    del seq_lens
    mcc = mcc_code.reshape(-1).astype(jnp.int32)
    tr = tr_type.reshape(-1).astype(jnp.int32)
    amt = amount.reshape(-1)
    out = _sc_encode(mcc, tr, amt, W_mcc, W_tr)
    return out.reshape(B, L, OUTD)


# speedup vs baseline: 2.5014x; 1.1118x over previous
import functools

import jax
import jax.numpy as jnp
from jax import lax
from jax.experimental import pallas as pl
from jax.experimental.pallas import tpu as pltpu
from jax.experimental.pallas import tpu_sc as plsc

B, L = 4096, 200
N = B * L
D = 16
OUTD = 2 * D + 1
NC, NS = 2, 16
NW = NC * NS
PER_W = N // NW
CHUNK = 512
NCHUNK = PER_W // CHUNK
LN2 = 0.6931471805599453


def _log1p_abs_signed(a):
    x = 1.0 + jnp.abs(a)
    u = lax.bitcast_convert_type(x, jnp.int32)
    un = u + (0x3F800000 - 0x3F3504F3)
    k = (un >> 23) - 127
    um = (un & 0x007FFFFF) + 0x3F3504F3
    m = lax.bitcast_convert_type(um, jnp.float32)
    t = (m - 1.0) / (m + 1.0)
    t2 = t * t
    p = 2.0 * t * (1.0 + t2 * (1.0 / 3.0 + t2 * (0.2 + t2 * (1.0 / 7.0 + t2 * (1.0 / 9.0)))))
    logx = k.astype(jnp.float32) * LN2 + p
    return jnp.sign(a) * logx


def _body(mcc_hbm, tr_hbm, amt_hbm, wm_hbm, wt_hbm, out_hbm,
          idxm0, idxm1, idxt0, idxt1, amt0, amt1,
          rows1_0, rows1_1, rows2_0, rows2_1, obuf0, obuf1,
          si0, si1, sg0, sg1, so0, so1):
    wid = lax.axis_index("s") * NC + lax.axis_index("c")
    base = wid * PER_W
    idxm = (idxm0, idxm1)
    idxt = (idxt0, idxt1)
    amtb = (amt0, amt1)
    rows1 = (rows1_0, rows1_1)
    rows2 = (rows2_0, rows2_1)
    obuf = (obuf0, obuf1)
    sem_i = (si0, si1)
    sem_g = (sg0, sg1)
    sem_o = (so0, so1)

    def start_idx(c, b):
        off = base + c * CHUNK
        pltpu.async_copy(mcc_hbm.at[pl.ds(off, CHUNK)], idxm[b], sem_i[b])
        pltpu.async_copy(tr_hbm.at[pl.ds(off, CHUNK)], idxt[b], sem_i[b])
        pltpu.async_copy(amt_hbm.at[pl.ds(off, CHUNK)], amtb[b], sem_i[b])

    def wait_idx(b):
        pltpu.make_async_copy(mcc_hbm.at[pl.ds(0, CHUNK)], idxm[b], sem_i[b]).wait()
        pltpu.make_async_copy(tr_hbm.at[pl.ds(0, CHUNK)], idxt[b], sem_i[b]).wait()
        pltpu.make_async_copy(amt_hbm.at[pl.ds(0, CHUNK)], amtb[b], sem_i[b]).wait()

    def start_gather(b):
        pltpu.async_copy(wm_hbm.at[idxm[b]], rows1[b], sem_g[b])
        pltpu.async_copy(wt_hbm.at[idxt[b]], rows2[b], sem_g[b])

    def wait_gather(b):
        pltpu.make_async_copy(wm_hbm.at[idxm[b]], rows1[b], sem_g[b]).wait()
        pltpu.make_async_copy(wt_hbm.at[idxt[b]], rows2[b], sem_g[b]).wait()

    def start_out(c, b):
        off = base + c * CHUNK
        pltpu.async_copy(obuf[b], out_hbm.at[pl.ds(off * OUTD, CHUNK * OUTD)], sem_o[b])

    def wait_out(b):
        pltpu.make_async_copy(obuf[b], out_hbm.at[pl.ds(0, CHUNK * OUTD)], sem_o[b]).wait()

    def interleave(b):
        r1, r2, ob, am = rows1[b], rows2[b], obuf[b], amtb[b]

        @plsc.parallel_loop(0, CHUNK // D, unroll=2)
        def _(i):
            r0 = i * D
            for j in range(D):
                r = r0 + j
                ob[pl.ds(r * OUTD, D)] = r1[r, :]
                ob[pl.ds(r * OUTD + D, D)] = r2[r, :]
            a = am[pl.ds(r0, D)]
            num = _log1p_abs_signed(a)
            idxs = (r0 + lax.iota(jnp.int32, D)) * OUTD + 2 * D
            plsc.store_scatter(ob, [idxs], num)

    def pipe_step(c, b, next_gather, idx_ahead, out_wait):
        wait_gather(b)
        nb = 1 - b
        if next_gather:
            wait_idx(nb)
            start_gather(nb)
        if out_wait:
            wait_out(b)
        interleave(b)
        if idx_ahead:
            start_idx(c + 2, b)
        start_out(c, b)

    start_idx(0, 0)
    start_idx(1, 1)
    wait_idx(0)
    start_gather(0)
    pipe_step(0, 0, True, True, False)
    pipe_step(1, 1, True, True, False)

    def pipe_k(k, carry):
        c = 2 * k
        pipe_step(c, 0, True, True, True)
        pipe_step(c + 1, 1, True, True, True)
        return carry

    lax.fori_loop(1, NCHUNK // 2 - 1, pipe_k, 0, unroll=False)
    pipe_step(NCHUNK - 2, 0, True, False, True)
    pipe_step(NCHUNK - 1, 1, False, False, True)
    wait_out(0)
    wait_out(1)


@jax.jit
def _sc_encode(mcc, tr, amt, wm, wt):
    mesh = plsc.VectorSubcoreMesh(core_axis_name="c", subcore_axis_name="s")
    f = pl.kernel(
        _body,
        out_type=jax.ShapeDtypeStruct((N * OUTD,), jnp.float32),
        mesh=mesh,
        compiler_params=pltpu.CompilerParams(
            needs_layout_passes=False, use_tc_tiling_on_sc=False),
        scratch_types=[
            pltpu.VMEM((CHUNK,), jnp.int32),
            pltpu.VMEM((CHUNK,), jnp.int32),
            pltpu.VMEM((CHUNK,), jnp.int32),
            pltpu.VMEM((CHUNK,), jnp.int32),
            pltpu.VMEM((CHUNK,), jnp.float32),
            pltpu.VMEM((CHUNK,), jnp.float32),
            pltpu.VMEM((CHUNK, D), jnp.float32),
            pltpu.VMEM((CHUNK, D), jnp.float32),
            pltpu.VMEM((CHUNK, D), jnp.float32),
            pltpu.VMEM((CHUNK, D), jnp.float32),
            pltpu.VMEM((CHUNK * OUTD,), jnp.float32),
            pltpu.VMEM((CHUNK * OUTD,), jnp.float32),
            pltpu.SemaphoreType.DMA,
            pltpu.SemaphoreType.DMA,
            pltpu.SemaphoreType.DMA,
            pltpu.SemaphoreType.DMA,
            pltpu.SemaphoreType.DMA,
            pltpu.SemaphoreType.DMA,
        ],
    )
    return f(mcc, tr, amt, wm, wt)


def kernel(mcc_code, tr_type, amount, seq_lens, W_mcc, W_tr):
    del seq_lens
    mcc = mcc_code.reshape(-1).astype(jnp.int32)
    tr = tr_type.reshape(-1).astype(jnp.int32)
    amt = amount.reshape(-1)
    out = _sc_encode(mcc, tr, amt, W_mcc, W_tr)
    return out.reshape(B, L, OUTD)


# speedup vs baseline: 2.5185x; 1.0068x over previous
import functools

import jax
import jax.numpy as jnp
from jax import lax
from jax.experimental import pallas as pl
from jax.experimental.pallas import tpu as pltpu
from jax.experimental.pallas import tpu_sc as plsc

B, L = 4096, 200
N = B * L
D = 16
OUTD = 2 * D + 1
NC, NS = 2, 16
NW = NC * NS
PER_W = N // NW
CHUNK = 800
NCHUNK = PER_W // CHUNK
LN2 = 0.6931471805599453


def _log1p_abs_signed(a):
    x = 1.0 + jnp.abs(a)
    u = lax.bitcast_convert_type(x, jnp.int32)
    un = u + (0x3F800000 - 0x3F3504F3)
    k = (un >> 23) - 127
    um = (un & 0x007FFFFF) + 0x3F3504F3
    m = lax.bitcast_convert_type(um, jnp.float32)
    t = (m - 1.0) / (m + 1.0)
    t2 = t * t
    p = 2.0 * t * (1.0 + t2 * (1.0 / 3.0 + t2 * (0.2 + t2 * (1.0 / 7.0 + t2 * (1.0 / 9.0)))))
    logx = k.astype(jnp.float32) * LN2 + p
    return jnp.sign(a) * logx


def _body(mcc_hbm, tr_hbm, amt_hbm, wm_hbm, wt_hbm, out_hbm,
          idxm0, idxm1, idxt0, idxt1, amt0, amt1,
          rows1_0, rows1_1, rows2_0, rows2_1, obuf0, obuf1,
          si0, si1, sg0, sg1, so0, so1):
    wid = lax.axis_index("s") * NC + lax.axis_index("c")
    base = wid * PER_W
    idxm = (idxm0, idxm1)
    idxt = (idxt0, idxt1)
    amtb = (amt0, amt1)
    rows1 = (rows1_0, rows1_1)
    rows2 = (rows2_0, rows2_1)
    obuf = (obuf0, obuf1)
    sem_i = (si0, si1)
    sem_g = (sg0, sg1)
    sem_o = (so0, so1)

    def start_idx(c, b):
        off = base + c * CHUNK
        pltpu.async_copy(mcc_hbm.at[pl.ds(off, CHUNK)], idxm[b], sem_i[b])
        pltpu.async_copy(tr_hbm.at[pl.ds(off, CHUNK)], idxt[b], sem_i[b])
        pltpu.async_copy(amt_hbm.at[pl.ds(off, CHUNK)], amtb[b], sem_i[b])

    def wait_idx(b):
        pltpu.make_async_copy(mcc_hbm.at[pl.ds(0, CHUNK)], idxm[b], sem_i[b]).wait()
        pltpu.make_async_copy(tr_hbm.at[pl.ds(0, CHUNK)], idxt[b], sem_i[b]).wait()
        pltpu.make_async_copy(amt_hbm.at[pl.ds(0, CHUNK)], amtb[b], sem_i[b]).wait()

    def start_gather(b):
        pltpu.async_copy(wm_hbm.at[idxm[b]], rows1[b], sem_g[b])
        pltpu.async_copy(wt_hbm.at[idxt[b]], rows2[b], sem_g[b])

    def wait_gather(b):
        pltpu.make_async_copy(wm_hbm.at[idxm[b]], rows1[b], sem_g[b]).wait()
        pltpu.make_async_copy(wt_hbm.at[idxt[b]], rows2[b], sem_g[b]).wait()

    def start_out(c, b):
        off = base + c * CHUNK
        pltpu.async_copy(obuf[b], out_hbm.at[pl.ds(off * OUTD, CHUNK * OUTD)], sem_o[b])

    def wait_out(b):
        pltpu.make_async_copy(obuf[b], out_hbm.at[pl.ds(0, CHUNK * OUTD)], sem_o[b]).wait()

    def interleave(b):
        r1, r2, ob, am = rows1[b], rows2[b], obuf[b], amtb[b]

        @plsc.parallel_loop(0, CHUNK // D, unroll=2)
        def _(i):
            r0 = i * D
            for j in range(D):
                r = r0 + j
                ob[pl.ds(r * OUTD, D)] = r1[r, :]
                ob[pl.ds(r * OUTD + D, D)] = r2[r, :]
            a = am[pl.ds(r0, D)]
            num = _log1p_abs_signed(a)
            idxs = (r0 + lax.iota(jnp.int32, D)) * OUTD + 2 * D
            plsc.store_scatter(ob, [idxs], num)

    def pipe_step(c, b, next_gather, idx_ahead, out_wait):
        wait_gather(b)
        nb = 1 - b
        if next_gather:
            wait_idx(nb)
            start_gather(nb)
        if out_wait:
            wait_out(b)
        interleave(b)
        if idx_ahead:
            start_idx(c + 2, b)
        start_out(c, b)

    start_idx(0, 0)
    start_idx(1, 1)
    wait_idx(0)
    start_gather(0)
    pipe_step(0, 0, True, True, False)
    pipe_step(1, 1, True, True, False)

    def pipe_k(k, carry):
        c = 2 * k
        pipe_step(c, 0, True, True, True)
        pipe_step(c + 1, 1, True, True, True)
        return carry

    lax.fori_loop(1, NCHUNK // 2 - 1, pipe_k, 0, unroll=False)
    pipe_step(NCHUNK - 2, 0, True, False, True)
    pipe_step(NCHUNK - 1, 1, False, False, True)
    wait_out(0)
    wait_out(1)


@jax.jit
def _sc_encode(mcc, tr, amt, wm, wt):
    mesh = plsc.VectorSubcoreMesh(core_axis_name="c", subcore_axis_name="s")
    f = pl.kernel(
        _body,
        out_type=jax.ShapeDtypeStruct((N * OUTD,), jnp.float32),
        mesh=mesh,
        compiler_params=pltpu.CompilerParams(
            needs_layout_passes=False, use_tc_tiling_on_sc=False),
        scratch_types=[
            pltpu.VMEM((CHUNK,), jnp.int32),
            pltpu.VMEM((CHUNK,), jnp.int32),
            pltpu.VMEM((CHUNK,), jnp.int32),
            pltpu.VMEM((CHUNK,), jnp.int32),
            pltpu.VMEM((CHUNK,), jnp.float32),
            pltpu.VMEM((CHUNK,), jnp.float32),
            pltpu.VMEM((CHUNK, D), jnp.float32),
            pltpu.VMEM((CHUNK, D), jnp.float32),
            pltpu.VMEM((CHUNK, D), jnp.float32),
            pltpu.VMEM((CHUNK, D), jnp.float32),
            pltpu.VMEM((CHUNK * OUTD,), jnp.float32),
            pltpu.VMEM((CHUNK * OUTD,), jnp.float32),
            pltpu.SemaphoreType.DMA,
            pltpu.SemaphoreType.DMA,
            pltpu.SemaphoreType.DMA,
            pltpu.SemaphoreType.DMA,
            pltpu.SemaphoreType.DMA,
            pltpu.SemaphoreType.DMA,
        ],
    )
    return f(mcc, tr, amt, wm, wt)


def kernel(mcc_code, tr_type, amount, seq_lens, W_mcc, W_tr):
    del seq_lens
    mcc = mcc_code.reshape(-1).astype(jnp.int32)
    tr = tr_type.reshape(-1).astype(jnp.int32)
    amt = amount.reshape(-1)
    out = _sc_encode(mcc, tr, amt, W_mcc, W_tr)
    return out.reshape(B, L, OUTD)


# speedup vs baseline: 2.5356x; 1.0068x over previous
import functools

import jax
import jax.numpy as jnp
from jax import lax
from jax.experimental import pallas as pl
from jax.experimental.pallas import tpu as pltpu
from jax.experimental.pallas import tpu_sc as plsc

B, L = 4096, 200
N = B * L
D = 16
OUTD = 2 * D + 1
NC, NS = 2, 16
NW = NC * NS
PER_W = N // NW
CHUNK = 800
NCHUNK = PER_W // CHUNK
LN2 = 0.6931471805599453


def _log1p_abs_signed(a):
    x = 1.0 + jnp.abs(a)
    u = lax.bitcast_convert_type(x, jnp.int32)
    un = u + (0x3F800000 - 0x3F3504F3)
    k = (un >> 23) - 127
    um = (un & 0x007FFFFF) + 0x3F3504F3
    m = lax.bitcast_convert_type(um, jnp.float32)
    t = (m - 1.0) / (m + 1.0)
    t2 = t * t
    p = 2.0 * t * (1.0 + t2 * (1.0 / 3.0 + t2 * (0.2 + t2 * (1.0 / 7.0 + t2 * (1.0 / 9.0)))))
    logx = k.astype(jnp.float32) * LN2 + p
    return jnp.sign(a) * logx


def _body(mcc_hbm, tr_hbm, amt_hbm, wm_hbm, wt_hbm, out_hbm,
          idxm0, idxm1, idxt0, idxt1, amt0, amt1,
          rows1_0, rows1_1, rows2_0, rows2_1, obuf0, obuf1,
          si0, si1, sg0, sg1, so0, so1):
    wid = lax.axis_index("s") * NC + lax.axis_index("c")
    base = wid * PER_W
    idxm = (idxm0, idxm1)
    idxt = (idxt0, idxt1)
    amtb = (amt0, amt1)
    rows1 = (rows1_0, rows1_1)
    rows2 = (rows2_0, rows2_1)
    obuf = (obuf0, obuf1)
    sem_i = (si0, si1)
    sem_g = (sg0, sg1)
    sem_o = (so0, so1)

    def start_idx(c, b):
        off = base + c * CHUNK
        pltpu.async_copy(mcc_hbm.at[pl.ds(off, CHUNK)], idxm[b], sem_i[b])
        pltpu.async_copy(tr_hbm.at[pl.ds(off, CHUNK)], idxt[b], sem_i[b])
        pltpu.async_copy(amt_hbm.at[pl.ds(off, CHUNK)], amtb[b], sem_i[b])

    def wait_idx(b):
        pltpu.make_async_copy(mcc_hbm.at[pl.ds(0, CHUNK)], idxm[b], sem_i[b]).wait()
        pltpu.make_async_copy(tr_hbm.at[pl.ds(0, CHUNK)], idxt[b], sem_i[b]).wait()
        pltpu.make_async_copy(amt_hbm.at[pl.ds(0, CHUNK)], amtb[b], sem_i[b]).wait()

    def start_gather(b):
        pltpu.async_copy(wm_hbm.at[idxm[b]], rows1[b], sem_g[b])
        pltpu.async_copy(wt_hbm.at[idxt[b]], rows2[b], sem_g[b])

    def wait_gather(b):
        pltpu.make_async_copy(wm_hbm.at[idxm[b]], rows1[b], sem_g[b]).wait()
        pltpu.make_async_copy(wt_hbm.at[idxt[b]], rows2[b], sem_g[b]).wait()

    def start_out(c, b):
        off = base + c * CHUNK
        pltpu.async_copy(obuf[b], out_hbm.at[pl.ds(off * OUTD, CHUNK * OUTD)], sem_o[b])

    def wait_out(b):
        pltpu.make_async_copy(obuf[b], out_hbm.at[pl.ds(0, CHUNK * OUTD)], sem_o[b]).wait()

    def interleave(b):
        r1, r2, ob, am = rows1[b], rows2[b], obuf[b], amtb[b]

        @plsc.parallel_loop(0, CHUNK // D, unroll=4)
        def _(i):
            r0 = i * D
            for j in range(D):
                r = r0 + j
                ob[pl.ds(r * OUTD, D)] = r1[r, :]
                ob[pl.ds(r * OUTD + D, D)] = r2[r, :]
            a = am[pl.ds(r0, D)]
            num = _log1p_abs_signed(a)
            idxs = (r0 + lax.iota(jnp.int32, D)) * OUTD + 2 * D
            plsc.store_scatter(ob, [idxs], num)

    def pipe_step(c, b, next_gather, idx_ahead, out_wait):
        wait_gather(b)
        nb = 1 - b
        if next_gather:
            wait_idx(nb)
            start_gather(nb)
        if out_wait:
            wait_out(b)
        interleave(b)
        if idx_ahead:
            start_idx(c + 2, b)
        start_out(c, b)

    start_idx(0, 0)
    start_idx(1, 1)
    wait_idx(0)
    start_gather(0)
    pipe_step(0, 0, True, True, False)
    pipe_step(1, 1, True, True, False)

    def pipe_k(k, carry):
        c = 2 * k
        pipe_step(c, 0, True, True, True)
        pipe_step(c + 1, 1, True, True, True)
        return carry

    lax.fori_loop(1, NCHUNK // 2 - 1, pipe_k, 0, unroll=False)
    pipe_step(NCHUNK - 2, 0, True, False, True)
    pipe_step(NCHUNK - 1, 1, False, False, True)
    wait_out(0)
    wait_out(1)


@jax.jit
def _sc_encode(mcc, tr, amt, wm, wt):
    mesh = plsc.VectorSubcoreMesh(core_axis_name="c", subcore_axis_name="s")
    f = pl.kernel(
        _body,
        out_type=jax.ShapeDtypeStruct((N * OUTD,), jnp.float32),
        mesh=mesh,
        compiler_params=pltpu.CompilerParams(
            needs_layout_passes=False, use_tc_tiling_on_sc=False),
        scratch_types=[
            pltpu.VMEM((CHUNK,), jnp.int32),
            pltpu.VMEM((CHUNK,), jnp.int32),
            pltpu.VMEM((CHUNK,), jnp.int32),
            pltpu.VMEM((CHUNK,), jnp.int32),
            pltpu.VMEM((CHUNK,), jnp.float32),
            pltpu.VMEM((CHUNK,), jnp.float32),
            pltpu.VMEM((CHUNK, D), jnp.float32),
            pltpu.VMEM((CHUNK, D), jnp.float32),
            pltpu.VMEM((CHUNK, D), jnp.float32),
            pltpu.VMEM((CHUNK, D), jnp.float32),
            pltpu.VMEM((CHUNK * OUTD,), jnp.float32),
            pltpu.VMEM((CHUNK * OUTD,), jnp.float32),
            pltpu.SemaphoreType.DMA,
            pltpu.SemaphoreType.DMA,
            pltpu.SemaphoreType.DMA,
            pltpu.SemaphoreType.DMA,
            pltpu.SemaphoreType.DMA,
            pltpu.SemaphoreType.DMA,
        ],
    )
    return f(mcc, tr, amt, wm, wt)


def kernel(mcc_code, tr_type, amount, seq_lens, W_mcc, W_tr):
    del seq_lens
    mcc = mcc_code.reshape(-1).astype(jnp.int32)
    tr = tr_type.reshape(-1).astype(jnp.int32)
    amt = amount.reshape(-1)
    out = _sc_encode(mcc, tr, amt, W_mcc, W_tr)
    return out.reshape(B, L, OUTD)
